# Initial kernel scaffold; baseline (speedup 1.0000x reference)
import jax
import jax.numpy as jnp
from jax.experimental import pallas as pl


def kernel(cad_vec, flag_vec, index_vec, key_padding_mask, W_cx, W_cy, W_sf, W_si):
    raise NotImplementedError("write your pallas kernel here")


# speedup vs baseline: 1.0617x; 1.0617x over previous
import functools

import jax
import jax.numpy as jnp
from jax import lax
from jax.experimental import pallas as pl
from jax.experimental.pallas import tpu as pltpu
from jax.experimental.pallas import tpu_sc as plsc

D = 64
NC, NS, L = 2, 16, 16
NW = NC * NS
CHUNK = 128


@functools.cache
def _sc_embed(n_tokens, v_xy, v_f, v_i):
    n_per_w = n_tokens // NW
    n_chunks = n_per_w // CHUNK
    mesh = plsc.VectorSubcoreMesh(core_axis_name="c", subcore_axis_name="s")

    @functools.partial(
        pl.kernel,
        out_type=jax.ShapeDtypeStruct((n_tokens, D), jnp.float32),
        mesh=mesh,
        compiler_params=pltpu.CompilerParams(use_tc_tiling_on_sc=False),
        scratch_types=[
            pltpu.VMEM((CHUNK,), jnp.int32),
            pltpu.VMEM((CHUNK,), jnp.int32),
            pltpu.VMEM((CHUNK,), jnp.int32),
            pltpu.VMEM((CHUNK,), jnp.int32),
            pltpu.VMEM((CHUNK,), jnp.int32),
            pltpu.VMEM((CHUNK, D), jnp.float32),
            pltpu.VMEM((CHUNK, D), jnp.float32),
            pltpu.VMEM((CHUNK, D), jnp.float32),
            pltpu.VMEM((CHUNK, D), jnp.float32),
            pltpu.SemaphoreType.DMA,
        ],
    )
    def k(x_hbm, y_hbm, fl_hbm, in_hbm, act_hbm, wcx, wcy, wsf, wsi, out_hbm,
          ix_v, iy_v, fl_v, in_v, act_v, rx, ry, rf, ri, sem):
        wid = lax.axis_index("s") * NC + lax.axis_index("c")
        w_base = wid * n_per_w

        def chunk_body(j, carry):
            base = w_base + j * CHUNK
            sl_hbm = pl.ds(base, CHUNK)
            pltpu.sync_copy(x_hbm.at[sl_hbm], ix_v)
            pltpu.sync_copy(y_hbm.at[sl_hbm], iy_v)
            pltpu.sync_copy(act_hbm.at[sl_hbm], act_v)
            pltpu.sync_copy(fl_hbm.at[sl_hbm], fl_v)
            pltpu.sync_copy(in_hbm.at[sl_hbm], in_v)
            for kk in range(CHUNK // L):
                sl = pl.ds(kk * L, L)
                a = act_v[sl]
                ix_v[sl] = ix_v[sl] * a
                iy_v[sl] = iy_v[sl] * a
            c1 = pltpu.async_copy(wcx.at[ix_v], rx, sem)
            c2 = pltpu.async_copy(wcy.at[iy_v], ry, sem)
            c3 = pltpu.async_copy(wsf.at[fl_v], rf, sem)
            c4 = pltpu.async_copy(wsi.at[in_v], ri, sem)
            c1.wait()
            c2.wait()
            c3.wait()
            c4.wait()

            def sum_body(r, c):
                for kk in range(D // L):
                    sl = pl.ds(kk * L, L)
                    rx[r, sl] = rx[r, sl] + ry[r, sl] + rf[r, sl] + ri[r, sl]
                return c

            lax.fori_loop(0, CHUNK, sum_body, 0)
            pltpu.sync_copy(rx, out_hbm.at[sl_hbm])
            return carry

        lax.fori_loop(0, n_chunks, chunk_body, 0)

    return k


def kernel(cad_vec, flag_vec, index_vec, key_padding_mask, W_cx, W_cy, W_sf,
           W_si):
    B, S = flag_vec.shape
    n = B * S
    x = cad_vec[:, :, 0].reshape(n)
    y = cad_vec[:, :, 1].reshape(n)
    fl = flag_vec.reshape(n)
    iv = index_vec.reshape(n)
    act = (~key_padding_mask).reshape(n).astype(jnp.int32)
    f = _sc_embed(n, W_cx.shape[0], W_sf.shape[0], W_si.shape[0])
    out = f(x, y, fl, iv, act, W_cx, W_cy, W_sf, W_si)
    return out.reshape(B, S, D)
